# baseline (speedup 1.0000x reference)
import functools

import jax
import jax.numpy as jnp
from jax import lax
from jax.experimental import pallas as pl
from jax.experimental.pallas import tpu as pltpu
from jax.experimental.pallas import tpu_sc as plsc

_N = 10000
_NP = 10112
_NOUT = 10112
_OPT = 632
_OLENS = (128, 128, 128, 128, 120)
_NCORE = 2
_NSUB = 16
_BM = 400
_CB = 64
_CW = 128

_MESH = plsc.VectorSubcoreMesh(core_axis_name="c", subcore_axis_name="s")


def _eprep_body(src_hbm, dst_hbm, srcq_hbm, dstq_hbm,
                src_v, dst_v, srcq_v, dstq_v):
    c = lax.axis_index("c")
    s = lax.axis_index("s")
    w = s * _NCORE + c
    e = src_hbm.shape[0]
    epw = e // (_NCORE * _NSUB)
    epad = ((epw + 15) // 16) * 16
    ngrp = epad // 16

    pltpu.sync_copy(src_hbm.at[pl.ds(w * epw, epw)], src_v.at[pl.ds(0, epw)])
    pltpu.sync_copy(dst_hbm.at[pl.ds(w * epw, epw)], dst_v.at[pl.ds(0, epw)])

    def grp(i, carry):
        s16 = src_v[pl.ds(i * 16, 16)]
        d16 = dst_v[pl.ds(i * 16, 16)]
        nosl = s16 != d16
        srcq_v[pl.ds(i * 16, 16)] = jnp.where(nosl, s16, _N)
        dstq_v[pl.ds(i * 16, 16)] = jnp.where(nosl, d16, _N)
        return carry
    lax.fori_loop(0, ngrp, grp, 0)

    pltpu.sync_copy(srcq_v.at[pl.ds(0, epw)], srcq_hbm.at[pl.ds(w * epw, epw)])
    pltpu.sync_copy(dstq_v.at[pl.ds(0, epw)], dstq_hbm.at[pl.ds(w * epw, epw)])


def _eprep_call(src, dst):
    e = src.shape[0]
    epw = e // (_NCORE * _NSUB)
    epad = ((epw + 15) // 16) * 16
    call = pl.kernel(
        _eprep_body,
        out_type=[jax.ShapeDtypeStruct((e,), jnp.int32),
                  jax.ShapeDtypeStruct((e,), jnp.int32)],
        mesh=_MESH,
        scratch_types=[
            pltpu.VMEM((epad,), jnp.int32),
            pltpu.VMEM((epad,), jnp.int32),
            pltpu.VMEM((epad,), jnp.int32),
            pltpu.VMEM((epad,), jnp.int32),
        ],
    )
    return call(src, dst)


def _prop_body(g0, g1, srcq, dstq, o0, o1,
               ibs0, ibs1, ibd0, ibd1, isrc_t, idst_t,
               rows0, rows1, rows_t, zrows, acc,
               sem_g0, sem_g1, sem_i0, sem_i1):
    c = lax.axis_index("c")
    s = lax.axis_index("s")
    e = srcq.shape[0]
    ept = e // _NSUB
    base = s * ept
    nfull = ept // _CB
    tail = ept - nfull * _CB

    ibs = (ibs0, ibs1)
    ibd = (ibd0, ibd1)
    rows = (rows0, rows1)
    sem_g = (sem_g0, sem_g1)
    sem_i = (sem_i0, sem_i1)

    z16 = jnp.zeros((16,), jnp.float32)

    def zrow(r, carry):
        for j in range(_CW // 16):
            zrows[r, pl.ds(j * 16, 16)] = z16
        return carry
    lax.fori_loop(0, 128, zrow, 0)

    def zpiece(t, carry):
        pltpu.sync_copy(zrows, acc.at[pl.ds(s * _OPT + t * 128, 128)])
        return carry
    lax.fori_loop(0, 4, zpiece, 0)
    pltpu.sync_copy(zrows.at[pl.ds(0, _OPT - 512)],
                    acc.at[pl.ds(s * _OPT + 512, _OPT - 512)])
    plsc.subcore_barrier()

    def gather_start(p, idxref, rowsref):
        @pl.when(c == 0)
        def _():
            pltpu.async_copy(g0.at[idxref], rowsref, sem_g[p])

        @pl.when(c == 1)
        def _():
            pltpu.async_copy(g1.at[idxref], rowsref, sem_g[p])

    def gather_wait(p, rowsref):
        pltpu.make_async_copy(g0.at[pl.ds(0, rowsref.shape[0])],
                              rowsref, sem_g[p]).wait()

    def idx_wait(p):
        pltpu.make_async_copy(srcq.at[pl.ds(0, _CB)], ibs[p], sem_i[p]).wait()
        pltpu.make_async_copy(dstq.at[pl.ds(0, _CB)], ibd[p], sem_i[p]).wait()

    def scat_start(p):
        pltpu.async_copy(rows[p], acc.at[ibd[p]], sem_i[p], add=True)

    def scat_wait(p):
        pltpu.make_async_copy(rows[p], acc.at[ibd[p]], sem_i[p]).wait()

    pltpu.sync_copy(srcq.at[pl.ds(base, _CB)], ibs0)
    pltpu.sync_copy(dstq.at[pl.ds(base, _CB)], ibd0)

    def half(j, p, first):
        q = 1 - p
        gather_start(p, ibs[p], rows[p])
        if not first:
            scat_wait(q)
        off = base + (j + 1) * _CB
        pltpu.sync_copy(srcq.at[pl.ds(off, _CB)], ibs[q])
        pltpu.sync_copy(dstq.at[pl.ds(off, _CB)], ibd[q])
        gather_wait(p, rows[p])
        scat_start(p)

    half(0, 0, True)

    def duo(j2, carry):
        j = j2 * 2 + 1
        half(j, 1, False)
        half(j + 1, 0, False)
        return carry
    lax.fori_loop(0, (nfull - 2) // 2, duo, 0)
    gather_start(1, ibs1, rows1)
    scat_wait(0)
    gather_wait(1, rows1)
    scat_start(1)
    scat_wait(1)

    if tail:
        off = base + nfull * _CB
        pltpu.sync_copy(srcq.at[pl.ds(off, tail)], isrc_t)
        pltpu.sync_copy(dstq.at[pl.ds(off, tail)], idst_t)
        gather_start(0, isrc_t, rows_t)
        pltpu.make_async_copy(g0.at[pl.ds(0, tail)], rows_t, sem_g0).wait()
        pltpu.sync_copy(rows_t, acc.at[idst_t], add=True)

    plsc.subcore_barrier()

    def opiece(t, carry):
        off = s * _OPT + t * 128
        pltpu.sync_copy(acc.at[pl.ds(off, 128)], zrows)

        @pl.when(c == 0)
        def _():
            pltpu.sync_copy(zrows, o0.at[pl.ds(off, 128)])

        @pl.when(c == 1)
        def _():
            pltpu.sync_copy(zrows, o1.at[pl.ds(off, 128)])
        return carry
    lax.fori_loop(0, 4, opiece, 0)
    lno = _OPT - 512
    offo = s * _OPT + 512
    pltpu.sync_copy(acc.at[pl.ds(offo, lno)], zrows.at[pl.ds(0, lno)])

    @pl.when(c == 0)
    def _():
        pltpu.sync_copy(zrows.at[pl.ds(0, lno)], o0.at[pl.ds(offo, lno)])

    @pl.when(c == 1)
    def _():
        pltpu.sync_copy(zrows.at[pl.ds(0, lno)], o1.at[pl.ds(offo, lno)])


def _prop_pair(g0, g1, srcq, dstq):
    e = srcq.shape[0]
    ept = e // _NSUB
    tail = ept - (ept // _CB) * _CB
    call = pl.kernel(
        _prop_body,
        out_type=[jax.ShapeDtypeStruct((_NOUT, _CW), jnp.float32),
                  jax.ShapeDtypeStruct((_NOUT, _CW), jnp.float32)],
        mesh=_MESH,
        scratch_types=[
            pltpu.VMEM((_CB,), jnp.int32),
            pltpu.VMEM((_CB,), jnp.int32),
            pltpu.VMEM((_CB,), jnp.int32),
            pltpu.VMEM((_CB,), jnp.int32),
            pltpu.VMEM((max(tail, 8),), jnp.int32),
            pltpu.VMEM((max(tail, 8),), jnp.int32),
            pltpu.VMEM((_CB, _CW), jnp.float32),
            pltpu.VMEM((_CB, _CW), jnp.float32),
            pltpu.VMEM((max(tail, 8), _CW), jnp.float32),
            pltpu.VMEM((128, _CW), jnp.float32),
            pltpu.VMEM_SHARED((_NP, _CW), jnp.float32),
            pltpu.SemaphoreType.DMA,
            pltpu.SemaphoreType.DMA,
            pltpu.SemaphoreType.DMA,
            pltpu.SemaphoreType.DMA,
        ],
    )
    return call(g0, g1, srcq, dstq)


def _prop_chunks(chunks, srcq, dstq):
    out = []
    for i in range(0, len(chunks), 2):
        o0, o1 = _prop_pair(chunks[i], chunks[i + 1], srcq, dstq)
        out.extend([o0, o1])
    return out


def _e0_body(deg_ref, x_ref, *out_refs):
    d = deg_ref[:, 0:1]
    dinv = jnp.where(d > 0, lax.rsqrt(d), 0.0)
    nc = x_ref.shape[1] // _CW
    for cch in range(nc):
        out_refs[cch][...] = dinv * x_ref[:, cch * _CW:(cch + 1) * _CW]
    out_refs[nc][...] = dinv
    out_refs[nc + 1][...] = dinv * dinv


def _e0(deg, x):
    n = x.shape[0]
    nc = x.shape[1] // _CW
    grid = (n // _BM,)
    cspec = pl.BlockSpec((_BM, _CW), lambda i: (i, 0))
    return pl.pallas_call(
        _e0_body,
        grid=grid,
        in_specs=[pl.BlockSpec((_BM, _CW), lambda i: (i, 0)),
                  pl.BlockSpec((_BM, x.shape[1]), lambda i: (i, 0))],
        out_specs=[cspec] * nc + [pl.BlockSpec((_BM, 1), lambda i: (i, 0))] * 2,
        out_shape=[jax.ShapeDtypeStruct((n, _CW), jnp.float32)] * nc
                  + [jax.ShapeDtypeStruct((n, 1), jnp.float32)] * 2,
    )(deg, x)


def _u1_body(nc, d2_ref, *refs):
    d2 = d2_ref[...]
    for c in range(nc):
        refs[nc + c][...] = -d2 * refs[c][...]


def _uscale(body, d2, chunk_lists):
    nc = len(chunk_lists[0])
    n = d2.shape[0]
    grid = (n // _BM,)
    cspec = pl.BlockSpec((_BM, _CW), lambda i: (i, 0))
    flat = [a for lst in chunk_lists for a in lst]
    return pl.pallas_call(
        functools.partial(body, nc),
        grid=grid,
        in_specs=[pl.BlockSpec((_BM, 1), lambda i: (i, 0))] + [cspec] * len(flat),
        out_specs=[cspec] * nc,
        out_shape=[jax.ShapeDtypeStruct((n, _CW), jnp.float32)] * nc,
    )(d2, *flat)


def _combine_body(nc, d_in, relu, emit_u, x_ref, dinv_ref, *refs):
    dinv = dinv_ref[...]
    parts = [x_ref[...]]
    for i in range(3 * nc):
        parts.append(dinv * refs[i][...])
    a = jnp.concatenate(parts, axis=1)
    v_ref = refs[3 * nc]
    b_ref = refs[3 * nc + 1]
    y = jnp.dot(a, v_ref[...], preferred_element_type=jnp.float32) + b_ref[...]
    if relu:
        y = jnp.maximum(y, 0.0)
    refs[3 * nc + 2][...] = y
    if emit_u:
        for cch in range(y.shape[1] // _CW):
            refs[3 * nc + 3 + cch][...] = dinv * y[:, cch * _CW:(cch + 1) * _CW]


def _combine(x, dinv, p1, p2, p3, w, b, relu, emit_u):
    n, d_in = x.shape
    d_out = w.shape[2]
    nc = d_in // _CW
    v = jnp.concatenate([w[0] - w[2], 3.0 * w[3] - w[1],
                         -2.0 * w[2], -4.0 * w[3]], axis=0)
    grid = (n // _BM,)
    cspec = pl.BlockSpec((_BM, _CW), lambda i: (i, 0))
    out_shape = [jax.ShapeDtypeStruct((n, d_out), jnp.float32)]
    out_specs = [pl.BlockSpec((_BM, d_out), lambda i: (i, 0))]
    if emit_u:
        out_shape += [jax.ShapeDtypeStruct((n, _CW), jnp.float32)] * (d_out // _CW)
        out_specs += [cspec] * (d_out // _CW)
    res = pl.pallas_call(
        functools.partial(_combine_body, nc, d_in, relu, emit_u),
        grid=grid,
        in_specs=[pl.BlockSpec((_BM, d_in), lambda i: (i, 0)),
                  pl.BlockSpec((_BM, 1), lambda i: (i, 0))]
                 + [cspec] * (3 * nc)
                 + [pl.BlockSpec((4 * d_in, d_out), lambda i: (0, 0)),
                    pl.BlockSpec((1, d_out), lambda i: (0, 0))],
        out_specs=out_specs,
        out_shape=out_shape,
    )(x, dinv, *p1, *p2, *p3, v, b.reshape(1, -1))
    return res if emit_u else res[0]


def _layer(x, dinv, dinv2, u0, srcq, dstq, w, b, relu, emit_u):
    p1 = _prop_chunks(u0, srcq, dstq)
    u1 = _uscale(_u1_body, dinv2, [p1])
    p2 = _prop_chunks(u1, srcq, dstq)
    u2 = _uscale(_u1_body, dinv2, [p2])
    p3 = _prop_chunks(u2, srcq, dstq)
    return _combine(x, dinv, p1, p2, p3, w, b, relu, emit_u)


def kernel(x, edge_index, W1, b1, W2, b2):
    src = edge_index[0]
    dst = edge_index[1]
    srcq, dstq = _eprep_call(src, dst)
    ones = jnp.ones((_NOUT, _CW), jnp.float32)
    deg, _unused = _prop_pair(ones, ones, srcq, srcq)
    *u0, dinv, dinv2 = _e0(deg, x)
    h, *u0p = _layer(x, dinv, dinv2, u0, src, dstq, W1, b1,
                     relu=True, emit_u=True)
    return _layer(h, dinv, dinv2, u0p, src, dstq, W2, b2,
                  relu=False, emit_u=False)


# speedup vs baseline: 1.2328x; 1.2328x over previous
import functools

import jax
import jax.numpy as jnp
from jax import lax
from jax.experimental import pallas as pl
from jax.experimental.pallas import tpu as pltpu
from jax.experimental.pallas import tpu_sc as plsc

_N = 10000
_NP = 10112
_NOUT = 10112
_OPT = 632
_OLENS = (128, 128, 128, 128, 120)
_NCORE = 2
_NSUB = 16
_BM = 400
_CB = 96
_CW = 128

_MESH = plsc.VectorSubcoreMesh(core_axis_name="c", subcore_axis_name="s")


def _eprep_body(src_hbm, dst_hbm, srcq_hbm, dstq_hbm,
                src_v, dst_v, srcq_v, dstq_v):
    c = lax.axis_index("c")
    s = lax.axis_index("s")
    w = s * _NCORE + c
    e = src_hbm.shape[0]
    epw = e // (_NCORE * _NSUB)
    epad = ((epw + 15) // 16) * 16
    ngrp = epad // 16

    pltpu.sync_copy(src_hbm.at[pl.ds(w * epw, epw)], src_v.at[pl.ds(0, epw)])
    pltpu.sync_copy(dst_hbm.at[pl.ds(w * epw, epw)], dst_v.at[pl.ds(0, epw)])

    def grp(i, carry):
        s16 = src_v[pl.ds(i * 16, 16)]
        d16 = dst_v[pl.ds(i * 16, 16)]
        nosl = s16 != d16
        srcq_v[pl.ds(i * 16, 16)] = jnp.where(nosl, s16, _N)
        dstq_v[pl.ds(i * 16, 16)] = jnp.where(nosl, d16, _N)
        return carry
    lax.fori_loop(0, ngrp, grp, 0)

    pltpu.sync_copy(srcq_v.at[pl.ds(0, epw)], srcq_hbm.at[pl.ds(w * epw, epw)])
    pltpu.sync_copy(dstq_v.at[pl.ds(0, epw)], dstq_hbm.at[pl.ds(w * epw, epw)])


def _eprep_call(src, dst):
    e = src.shape[0]
    epw = e // (_NCORE * _NSUB)
    epad = ((epw + 15) // 16) * 16
    call = pl.kernel(
        _eprep_body,
        out_type=[jax.ShapeDtypeStruct((e,), jnp.int32),
                  jax.ShapeDtypeStruct((e,), jnp.int32)],
        mesh=_MESH,
        scratch_types=[
            pltpu.VMEM((epad,), jnp.int32),
            pltpu.VMEM((epad,), jnp.int32),
            pltpu.VMEM((epad,), jnp.int32),
            pltpu.VMEM((epad,), jnp.int32),
        ],
    )
    return call(src, dst)


def _prop_body(g0, g1, srcq, dstq, o0, o1,
               ibs0, ibs1, ibd0, ibd1, isrc_t, idst_t,
               rows0, rows1, rows_t, zrows, acc,
               sem_g0, sem_g1, sem_i0, sem_i1):
    c = lax.axis_index("c")
    s = lax.axis_index("s")
    e = srcq.shape[0]
    ept = e // _NSUB
    base = s * ept
    nfull = ept // _CB
    tail = ept - nfull * _CB

    ibs = (ibs0, ibs1)
    ibd = (ibd0, ibd1)
    rows = (rows0, rows1)
    sem_g = (sem_g0, sem_g1)
    sem_i = (sem_i0, sem_i1)

    z16 = jnp.zeros((16,), jnp.float32)

    def zrow(r, carry):
        for j in range(_CW // 16):
            zrows[r, pl.ds(j * 16, 16)] = z16
        return carry
    lax.fori_loop(0, 128, zrow, 0)

    def zpiece(t, carry):
        pltpu.sync_copy(zrows, acc.at[pl.ds(s * _OPT + t * 128, 128)])
        return carry
    lax.fori_loop(0, 4, zpiece, 0)
    pltpu.sync_copy(zrows.at[pl.ds(0, _OPT - 512)],
                    acc.at[pl.ds(s * _OPT + 512, _OPT - 512)])
    plsc.subcore_barrier()

    def gather_start(p, idxref, rowsref):
        @pl.when(c == 0)
        def _():
            pltpu.async_copy(g0.at[idxref], rowsref, sem_g[p])

        @pl.when(c == 1)
        def _():
            pltpu.async_copy(g1.at[idxref], rowsref, sem_g[p])

    def gather_wait(p, rowsref):
        pltpu.make_async_copy(g0.at[pl.ds(0, rowsref.shape[0])],
                              rowsref, sem_g[p]).wait()

    def idx_wait(p):
        pltpu.make_async_copy(srcq.at[pl.ds(0, _CB)], ibs[p], sem_i[p]).wait()
        pltpu.make_async_copy(dstq.at[pl.ds(0, _CB)], ibd[p], sem_i[p]).wait()

    def scat_start(p):
        pltpu.async_copy(rows[p], acc.at[ibd[p]], sem_i[p], add=True)

    def scat_wait(p):
        pltpu.make_async_copy(rows[p], acc.at[ibd[p]], sem_i[p]).wait()

    pltpu.sync_copy(srcq.at[pl.ds(base, _CB)], ibs0)
    pltpu.sync_copy(dstq.at[pl.ds(base, _CB)], ibd0)

    def half(j, p, first):
        q = 1 - p
        gather_start(p, ibs[p], rows[p])
        if not first:
            scat_wait(q)
        off = base + (j + 1) * _CB
        pltpu.sync_copy(srcq.at[pl.ds(off, _CB)], ibs[q])
        pltpu.sync_copy(dstq.at[pl.ds(off, _CB)], ibd[q])
        gather_wait(p, rows[p])
        scat_start(p)

    half(0, 0, True)

    def duo(j2, carry):
        j = j2 * 2 + 1
        half(j, 1, False)
        half(j + 1, 0, False)
        return carry
    lax.fori_loop(0, (nfull - 2) // 2, duo, 0)
    gather_start(1, ibs1, rows1)
    scat_wait(0)
    gather_wait(1, rows1)
    scat_start(1)
    scat_wait(1)

    if tail:
        off = base + nfull * _CB
        pltpu.sync_copy(srcq.at[pl.ds(off, tail)], isrc_t)
        pltpu.sync_copy(dstq.at[pl.ds(off, tail)], idst_t)
        gather_start(0, isrc_t, rows_t)
        pltpu.make_async_copy(g0.at[pl.ds(0, tail)], rows_t, sem_g0).wait()
        pltpu.sync_copy(rows_t, acc.at[idst_t], add=True)

    plsc.subcore_barrier()

    def opiece(t, carry):
        off = s * _OPT + t * 128
        pltpu.sync_copy(acc.at[pl.ds(off, 128)], zrows)

        @pl.when(c == 0)
        def _():
            pltpu.sync_copy(zrows, o0.at[pl.ds(off, 128)])

        @pl.when(c == 1)
        def _():
            pltpu.sync_copy(zrows, o1.at[pl.ds(off, 128)])
        return carry
    lax.fori_loop(0, 4, opiece, 0)
    lno = _OPT - 512
    offo = s * _OPT + 512
    pltpu.sync_copy(acc.at[pl.ds(offo, lno)], zrows.at[pl.ds(0, lno)])

    @pl.when(c == 0)
    def _():
        pltpu.sync_copy(zrows.at[pl.ds(0, lno)], o0.at[pl.ds(offo, lno)])

    @pl.when(c == 1)
    def _():
        pltpu.sync_copy(zrows.at[pl.ds(0, lno)], o1.at[pl.ds(offo, lno)])


def _prop_pair(g0, g1, srcq, dstq):
    e = srcq.shape[0]
    ept = e // _NSUB
    tail = ept - (ept // _CB) * _CB
    call = pl.kernel(
        _prop_body,
        out_type=[jax.ShapeDtypeStruct((_NOUT, _CW), jnp.float32),
                  jax.ShapeDtypeStruct((_NOUT, _CW), jnp.float32)],
        mesh=_MESH,
        scratch_types=[
            pltpu.VMEM((_CB,), jnp.int32),
            pltpu.VMEM((_CB,), jnp.int32),
            pltpu.VMEM((_CB,), jnp.int32),
            pltpu.VMEM((_CB,), jnp.int32),
            pltpu.VMEM((max(tail, 8),), jnp.int32),
            pltpu.VMEM((max(tail, 8),), jnp.int32),
            pltpu.VMEM((_CB, _CW), jnp.float32),
            pltpu.VMEM((_CB, _CW), jnp.float32),
            pltpu.VMEM((max(tail, 8), _CW), jnp.float32),
            pltpu.VMEM((128, _CW), jnp.float32),
            pltpu.VMEM_SHARED((_NP, _CW), jnp.float32),
            pltpu.SemaphoreType.DMA,
            pltpu.SemaphoreType.DMA,
            pltpu.SemaphoreType.DMA,
            pltpu.SemaphoreType.DMA,
        ],
    )
    return call(g0, g1, srcq, dstq)


def _prop_chunks(chunks, srcq, dstq):
    out = []
    for i in range(0, len(chunks), 2):
        o0, o1 = _prop_pair(chunks[i], chunks[i + 1], srcq, dstq)
        out.extend([o0, o1])
    return out


def _e0_body(deg_ref, x_ref, *out_refs):
    d = deg_ref[:, 0:1]
    dinv = jnp.where(d > 0, lax.rsqrt(d), 0.0)
    nc = x_ref.shape[1] // _CW
    for cch in range(nc):
        out_refs[cch][...] = dinv * x_ref[:, cch * _CW:(cch + 1) * _CW]
    out_refs[nc][...] = dinv
    out_refs[nc + 1][...] = dinv * dinv


def _e0(deg, x):
    n = x.shape[0]
    nc = x.shape[1] // _CW
    grid = (n // _BM,)
    cspec = pl.BlockSpec((_BM, _CW), lambda i: (i, 0))
    return pl.pallas_call(
        _e0_body,
        grid=grid,
        in_specs=[pl.BlockSpec((_BM, _CW), lambda i: (i, 0)),
                  pl.BlockSpec((_BM, x.shape[1]), lambda i: (i, 0))],
        out_specs=[cspec] * nc + [pl.BlockSpec((_BM, 1), lambda i: (i, 0))] * 2,
        out_shape=[jax.ShapeDtypeStruct((n, _CW), jnp.float32)] * nc
                  + [jax.ShapeDtypeStruct((n, 1), jnp.float32)] * 2,
    )(deg, x)


def _u1_body(nc, d2_ref, *refs):
    d2 = d2_ref[...]
    for c in range(nc):
        refs[nc + c][...] = -d2 * refs[c][...]


def _uscale(body, d2, chunk_lists):
    nc = len(chunk_lists[0])
    n = d2.shape[0]
    grid = (n // _BM,)
    cspec = pl.BlockSpec((_BM, _CW), lambda i: (i, 0))
    flat = [a for lst in chunk_lists for a in lst]
    return pl.pallas_call(
        functools.partial(body, nc),
        grid=grid,
        in_specs=[pl.BlockSpec((_BM, 1), lambda i: (i, 0))] + [cspec] * len(flat),
        out_specs=[cspec] * nc,
        out_shape=[jax.ShapeDtypeStruct((n, _CW), jnp.float32)] * nc,
    )(d2, *flat)


def _combine_body(nc, d_in, relu, emit_u, x_ref, dinv_ref, *refs):
    dinv = dinv_ref[...]
    parts = [x_ref[...]]
    for i in range(3 * nc):
        parts.append(dinv * refs[i][...])
    a = jnp.concatenate(parts, axis=1)
    v_ref = refs[3 * nc]
    b_ref = refs[3 * nc + 1]
    y = jnp.dot(a, v_ref[...], preferred_element_type=jnp.float32) + b_ref[...]
    if relu:
        y = jnp.maximum(y, 0.0)
    refs[3 * nc + 2][...] = y
    if emit_u:
        for cch in range(y.shape[1] // _CW):
            refs[3 * nc + 3 + cch][...] = dinv * y[:, cch * _CW:(cch + 1) * _CW]


def _combine(x, dinv, p1, p2, p3, w, b, relu, emit_u):
    n, d_in = x.shape
    d_out = w.shape[2]
    nc = d_in // _CW
    v = jnp.concatenate([w[0] - w[2], 3.0 * w[3] - w[1],
                         -2.0 * w[2], -4.0 * w[3]], axis=0)
    grid = (n // _BM,)
    cspec = pl.BlockSpec((_BM, _CW), lambda i: (i, 0))
    out_shape = [jax.ShapeDtypeStruct((n, d_out), jnp.float32)]
    out_specs = [pl.BlockSpec((_BM, d_out), lambda i: (i, 0))]
    if emit_u:
        out_shape += [jax.ShapeDtypeStruct((n, _CW), jnp.float32)] * (d_out // _CW)
        out_specs += [cspec] * (d_out // _CW)
    res = pl.pallas_call(
        functools.partial(_combine_body, nc, d_in, relu, emit_u),
        grid=grid,
        in_specs=[pl.BlockSpec((_BM, d_in), lambda i: (i, 0)),
                  pl.BlockSpec((_BM, 1), lambda i: (i, 0))]
                 + [cspec] * (3 * nc)
                 + [pl.BlockSpec((4 * d_in, d_out), lambda i: (0, 0)),
                    pl.BlockSpec((1, d_out), lambda i: (0, 0))],
        out_specs=out_specs,
        out_shape=out_shape,
    )(x, dinv, *p1, *p2, *p3, v, b.reshape(1, -1))
    return res if emit_u else res[0]


def _layer(x, dinv, dinv2, u0, srcq, dstq, w, b, relu, emit_u):
    p1 = _prop_chunks(u0, srcq, dstq)
    u1 = _uscale(_u1_body, dinv2, [p1])
    p2 = _prop_chunks(u1, srcq, dstq)
    u2 = _uscale(_u1_body, dinv2, [p2])
    p3 = _prop_chunks(u2, srcq, dstq)
    return _combine(x, dinv, p1, p2, p3, w, b, relu, emit_u)


def kernel(x, edge_index, W1, b1, W2, b2):
    src = edge_index[0]
    dst = edge_index[1]
    srcq, dstq = _eprep_call(src, dst)
    ones = jnp.ones((_NOUT, _CW), jnp.float32)
    deg, _unused = _prop_pair(ones, ones, srcq, srcq)
    *u0, dinv, dinv2 = _e0(deg, x)
    h, *u0p = _layer(x, dinv, dinv2, u0, src, dstq, W1, b1,
                     relu=True, emit_u=True)
    return _layer(h, dinv, dinv2, u0p, src, dstq, W2, b2,
                  relu=False, emit_u=False)


# speedup vs baseline: 1.3217x; 1.0721x over previous
import functools

import jax
import jax.numpy as jnp
from jax import lax
from jax.experimental import pallas as pl
from jax.experimental.pallas import tpu as pltpu
from jax.experimental.pallas import tpu_sc as plsc

_N = 10000
_NP = 10112
_NOUT = 10112
_OPT = 632
_OLENS = (128, 128, 128, 128, 120)
_NCORE = 2
_NSUB = 16
_BM = 400
_CB = 96
_CW = 128

_MESH = plsc.VectorSubcoreMesh(core_axis_name="c", subcore_axis_name="s")


def _degprep_body(src_hbm, dst_hbm, dstq_hbm, deg0_hbm, deg1_hbm,
                  src_v, dst_v, dstq_v, isrc, ones_v, zrows, acc):
    c = lax.axis_index("c")
    s = lax.axis_index("s")
    w = s * _NCORE + c
    e = src_hbm.shape[0]
    epw = e // (_NCORE * _NSUB)
    epad = ((epw + 15) // 16) * 16
    ngrp = epad // 16
    iota16 = lax.iota(jnp.int32, 16)
    z16 = jnp.zeros((16,), jnp.float32)
    one16 = jnp.ones((16,), jnp.float32)

    def zrow(r, carry):
        for j in range(_CW // 16):
            zrows[r, pl.ds(j * 16, 16)] = z16
        return carry
    lax.fori_loop(0, 128, zrow, 0)

    def orow(r, carry):
        for j in range(_CW // 16):
            ones_v[r, pl.ds(j * 16, 16)] = one16
        return carry
    lax.fori_loop(0, _CB, orow, 0)

    def zpiece(t, carry):
        pltpu.sync_copy(zrows, acc.at[pl.ds(s * _OPT + t * 128, 128)])
        return carry
    lax.fori_loop(0, 4, zpiece, 0)
    pltpu.sync_copy(zrows.at[pl.ds(0, _OPT - 512)],
                    acc.at[pl.ds(s * _OPT + 512, _OPT - 512)])

    pltpu.sync_copy(src_hbm.at[pl.ds(w * epw, epw)], src_v.at[pl.ds(0, epw)])
    pltpu.sync_copy(dst_hbm.at[pl.ds(w * epw, epw)], dst_v.at[pl.ds(0, epw)])
    plsc.subcore_barrier()

    def grp(i, carry):
        s16 = src_v[pl.ds(i * 16, 16)]
        d16 = dst_v[pl.ds(i * 16, 16)]
        valid = (i * 16 + iota16) < epw
        nosl = (s16 != d16) & valid
        src_v[pl.ds(i * 16, 16)] = jnp.where(nosl, s16, _N)
        dstq_v[pl.ds(i * 16, 16)] = jnp.where(nosl, d16, _N)
        return carry
    lax.fori_loop(0, ngrp, grp, 0)

    pltpu.sync_copy(dstq_v.at[pl.ds(0, epw)], dstq_hbm.at[pl.ds(w * epw, epw)])

    nch = epad // _CB

    def chunk(j, carry):
        for g in range(_CB // 16):
            isrc[pl.ds(g * 16, 16)] = src_v[pl.ds(j * _CB + g * 16, 16)]
        pltpu.sync_copy(ones_v, acc.at[isrc], add=True)
        return carry
    lax.fori_loop(0, nch, chunk, 0)
    rem = epad - nch * _CB
    if rem:
        for g in range(rem // 16):
            isrc[pl.ds(g * 16, 16)] = src_v[pl.ds(nch * _CB + g * 16, 16)]
        for g in range(rem // 16, _CB // 16):
            isrc[pl.ds(g * 16, 16)] = jnp.full((16,), _N, jnp.int32)
        pltpu.sync_copy(ones_v, acc.at[isrc], add=True)

    plsc.subcore_barrier()

    def opiece(t, carry):
        off = s * _OPT + t * 128
        pltpu.sync_copy(acc.at[pl.ds(off, 128)], zrows)

        @pl.when(c == 0)
        def _():
            pltpu.sync_copy(zrows, deg0_hbm.at[pl.ds(off, 128)])

        @pl.when(c == 1)
        def _():
            pltpu.sync_copy(zrows, deg1_hbm.at[pl.ds(off, 128)])
        return carry
    lax.fori_loop(0, 4, opiece, 0)
    lno = _OPT - 512
    offo = s * _OPT + 512
    pltpu.sync_copy(acc.at[pl.ds(offo, lno)], zrows.at[pl.ds(0, lno)])

    @pl.when(c == 0)
    def _():
        pltpu.sync_copy(zrows.at[pl.ds(0, lno)], deg0_hbm.at[pl.ds(offo, lno)])

    @pl.when(c == 1)
    def _():
        pltpu.sync_copy(zrows.at[pl.ds(0, lno)], deg1_hbm.at[pl.ds(offo, lno)])


def _degprep_call(src, dst):
    e = src.shape[0]
    epw = e // (_NCORE * _NSUB)
    epad = ((epw + 15) // 16) * 16
    call = pl.kernel(
        _degprep_body,
        out_type=[jax.ShapeDtypeStruct((e,), jnp.int32),
                  jax.ShapeDtypeStruct((_NOUT, _CW), jnp.float32),
                  jax.ShapeDtypeStruct((_NOUT, _CW), jnp.float32)],
        mesh=_MESH,
        scratch_types=[
            pltpu.VMEM((epad,), jnp.int32),
            pltpu.VMEM((epad,), jnp.int32),
            pltpu.VMEM((epad,), jnp.int32),
            pltpu.VMEM((_CB,), jnp.int32),
            pltpu.VMEM((_CB, _CW), jnp.float32),
            pltpu.VMEM((128, _CW), jnp.float32),
            pltpu.VMEM_SHARED((_NP, _CW), jnp.float32),
        ],
    )
    return call(src, dst)


def _prop_body(g0, g1, srcq, dstq, o0, o1,
               ibs0, ibs1, ibd0, ibd1, isrc_t, idst_t,
               rows0, rows1, rows_t, zrows, acc,
               sem_g0, sem_g1, sem_i0, sem_i1):
    c = lax.axis_index("c")
    s = lax.axis_index("s")
    e = srcq.shape[0]
    ept = e // _NSUB
    base = s * ept
    nfull = ept // _CB
    tail = ept - nfull * _CB

    ibs = (ibs0, ibs1)
    ibd = (ibd0, ibd1)
    rows = (rows0, rows1)
    sem_g = (sem_g0, sem_g1)
    sem_i = (sem_i0, sem_i1)

    z16 = jnp.zeros((16,), jnp.float32)

    def zrow(r, carry):
        for j in range(_CW // 16):
            zrows[r, pl.ds(j * 16, 16)] = z16
        return carry
    lax.fori_loop(0, 128, zrow, 0)

    def zpiece(t, carry):
        pltpu.sync_copy(zrows, acc.at[pl.ds(s * _OPT + t * 128, 128)])
        return carry
    lax.fori_loop(0, 4, zpiece, 0)
    pltpu.sync_copy(zrows.at[pl.ds(0, _OPT - 512)],
                    acc.at[pl.ds(s * _OPT + 512, _OPT - 512)])
    plsc.subcore_barrier()

    def gather_start(p, idxref, rowsref):
        @pl.when(c == 0)
        def _():
            pltpu.async_copy(g0.at[idxref], rowsref, sem_g[p])

        @pl.when(c == 1)
        def _():
            pltpu.async_copy(g1.at[idxref], rowsref, sem_g[p])

    def gather_wait(p, rowsref):
        pltpu.make_async_copy(g0.at[pl.ds(0, rowsref.shape[0])],
                              rowsref, sem_g[p]).wait()

    def idx_wait(p):
        pltpu.make_async_copy(srcq.at[pl.ds(0, _CB)], ibs[p], sem_i[p]).wait()
        pltpu.make_async_copy(dstq.at[pl.ds(0, _CB)], ibd[p], sem_i[p]).wait()

    def scat_start(p):
        pltpu.async_copy(rows[p], acc.at[ibd[p]], sem_i[p], add=True)

    def scat_wait(p):
        pltpu.make_async_copy(rows[p], acc.at[ibd[p]], sem_i[p]).wait()

    pltpu.sync_copy(srcq.at[pl.ds(base, _CB)], ibs0)
    pltpu.sync_copy(dstq.at[pl.ds(base, _CB)], ibd0)

    def half(j, p, first):
        q = 1 - p
        gather_start(p, ibs[p], rows[p])
        if not first:
            scat_wait(q)
        off = base + (j + 1) * _CB
        pltpu.sync_copy(srcq.at[pl.ds(off, _CB)], ibs[q])
        pltpu.sync_copy(dstq.at[pl.ds(off, _CB)], ibd[q])
        gather_wait(p, rows[p])
        scat_start(p)

    half(0, 0, True)

    def duo(j2, carry):
        j = j2 * 2 + 1
        half(j, 1, False)
        half(j + 1, 0, False)
        return carry
    lax.fori_loop(0, (nfull - 2) // 2, duo, 0)
    gather_start(1, ibs1, rows1)
    scat_wait(0)
    gather_wait(1, rows1)
    scat_start(1)
    scat_wait(1)

    if tail:
        off = base + nfull * _CB
        pltpu.sync_copy(srcq.at[pl.ds(off, tail)], isrc_t)
        pltpu.sync_copy(dstq.at[pl.ds(off, tail)], idst_t)
        gather_start(0, isrc_t, rows_t)
        pltpu.make_async_copy(g0.at[pl.ds(0, tail)], rows_t, sem_g0).wait()
        pltpu.sync_copy(rows_t, acc.at[idst_t], add=True)

    plsc.subcore_barrier()

    def opiece(t, carry):
        off = s * _OPT + t * 128
        pltpu.sync_copy(acc.at[pl.ds(off, 128)], zrows)

        @pl.when(c == 0)
        def _():
            pltpu.sync_copy(zrows, o0.at[pl.ds(off, 128)])

        @pl.when(c == 1)
        def _():
            pltpu.sync_copy(zrows, o1.at[pl.ds(off, 128)])
        return carry
    lax.fori_loop(0, 4, opiece, 0)
    lno = _OPT - 512
    offo = s * _OPT + 512
    pltpu.sync_copy(acc.at[pl.ds(offo, lno)], zrows.at[pl.ds(0, lno)])

    @pl.when(c == 0)
    def _():
        pltpu.sync_copy(zrows.at[pl.ds(0, lno)], o0.at[pl.ds(offo, lno)])

    @pl.when(c == 1)
    def _():
        pltpu.sync_copy(zrows.at[pl.ds(0, lno)], o1.at[pl.ds(offo, lno)])


def _prop_pair(g0, g1, srcq, dstq):
    e = srcq.shape[0]
    ept = e // _NSUB
    tail = ept - (ept // _CB) * _CB
    call = pl.kernel(
        _prop_body,
        out_type=[jax.ShapeDtypeStruct((_NOUT, _CW), jnp.float32),
                  jax.ShapeDtypeStruct((_NOUT, _CW), jnp.float32)],
        mesh=_MESH,
        scratch_types=[
            pltpu.VMEM((_CB,), jnp.int32),
            pltpu.VMEM((_CB,), jnp.int32),
            pltpu.VMEM((_CB,), jnp.int32),
            pltpu.VMEM((_CB,), jnp.int32),
            pltpu.VMEM((max(tail, 8),), jnp.int32),
            pltpu.VMEM((max(tail, 8),), jnp.int32),
            pltpu.VMEM((_CB, _CW), jnp.float32),
            pltpu.VMEM((_CB, _CW), jnp.float32),
            pltpu.VMEM((max(tail, 8), _CW), jnp.float32),
            pltpu.VMEM((128, _CW), jnp.float32),
            pltpu.VMEM_SHARED((_NP, _CW), jnp.float32),
            pltpu.SemaphoreType.DMA,
            pltpu.SemaphoreType.DMA,
            pltpu.SemaphoreType.DMA,
            pltpu.SemaphoreType.DMA,
        ],
    )
    return call(g0, g1, srcq, dstq)


def _prop_chunks(chunks, srcq, dstq):
    out = []
    for i in range(0, len(chunks), 2):
        o0, o1 = _prop_pair(chunks[i], chunks[i + 1], srcq, dstq)
        out.extend([o0, o1])
    return out


def _e0_body(deg0_ref, deg1_ref, x_ref, *out_refs):
    d = deg0_ref[:, 0:1] + deg1_ref[:, 0:1]
    dinv = jnp.where(d > 0, lax.rsqrt(d), 0.0)
    nc = x_ref.shape[1] // _CW
    for cch in range(nc):
        out_refs[cch][...] = dinv * x_ref[:, cch * _CW:(cch + 1) * _CW]
    out_refs[nc][...] = dinv
    out_refs[nc + 1][...] = dinv * dinv


def _e0(deg0, deg1, x):
    n = x.shape[0]
    nc = x.shape[1] // _CW
    grid = (n // _BM,)
    cspec = pl.BlockSpec((_BM, _CW), lambda i: (i, 0))
    return pl.pallas_call(
        _e0_body,
        grid=grid,
        in_specs=[pl.BlockSpec((_BM, _CW), lambda i: (i, 0)),
                  pl.BlockSpec((_BM, _CW), lambda i: (i, 0)),
                  pl.BlockSpec((_BM, x.shape[1]), lambda i: (i, 0))],
        out_specs=[cspec] * nc + [pl.BlockSpec((_BM, 1), lambda i: (i, 0))] * 2,
        out_shape=[jax.ShapeDtypeStruct((n, _CW), jnp.float32)] * nc
                  + [jax.ShapeDtypeStruct((n, 1), jnp.float32)] * 2,
    )(deg0, deg1, x)


def _u1_body(nc, d2_ref, *refs):
    d2 = d2_ref[...]
    for c in range(nc):
        refs[nc + c][...] = -d2 * refs[c][...]


def _uscale(body, d2, chunk_lists):
    nc = len(chunk_lists[0])
    n = d2.shape[0]
    grid = (n // _BM,)
    cspec = pl.BlockSpec((_BM, _CW), lambda i: (i, 0))
    flat = [a for lst in chunk_lists for a in lst]
    return pl.pallas_call(
        functools.partial(body, nc),
        grid=grid,
        in_specs=[pl.BlockSpec((_BM, 1), lambda i: (i, 0))] + [cspec] * len(flat),
        out_specs=[cspec] * nc,
        out_shape=[jax.ShapeDtypeStruct((n, _CW), jnp.float32)] * nc,
    )(d2, *flat)


def _combine_body(nc, d_in, relu, emit_u, x_ref, dinv_ref, *refs):
    dinv = dinv_ref[...]
    parts = [x_ref[...]]
    for i in range(3 * nc):
        parts.append(dinv * refs[i][...])
    a = jnp.concatenate(parts, axis=1)
    v_ref = refs[3 * nc]
    b_ref = refs[3 * nc + 1]
    y = jnp.dot(a, v_ref[...], preferred_element_type=jnp.float32) + b_ref[...]
    if relu:
        y = jnp.maximum(y, 0.0)
    refs[3 * nc + 2][...] = y
    if emit_u:
        for cch in range(y.shape[1] // _CW):
            refs[3 * nc + 3 + cch][...] = dinv * y[:, cch * _CW:(cch + 1) * _CW]


def _combine(x, dinv, p1, p2, p3, w, b, relu, emit_u):
    n, d_in = x.shape
    d_out = w.shape[2]
    nc = d_in // _CW
    v = jnp.concatenate([w[0] - w[2], 3.0 * w[3] - w[1],
                         -2.0 * w[2], -4.0 * w[3]], axis=0)
    grid = (n // _BM,)
    cspec = pl.BlockSpec((_BM, _CW), lambda i: (i, 0))
    out_shape = [jax.ShapeDtypeStruct((n, d_out), jnp.float32)]
    out_specs = [pl.BlockSpec((_BM, d_out), lambda i: (i, 0))]
    if emit_u:
        out_shape += [jax.ShapeDtypeStruct((n, _CW), jnp.float32)] * (d_out // _CW)
        out_specs += [cspec] * (d_out // _CW)
    res = pl.pallas_call(
        functools.partial(_combine_body, nc, d_in, relu, emit_u),
        grid=grid,
        in_specs=[pl.BlockSpec((_BM, d_in), lambda i: (i, 0)),
                  pl.BlockSpec((_BM, 1), lambda i: (i, 0))]
                 + [cspec] * (3 * nc)
                 + [pl.BlockSpec((4 * d_in, d_out), lambda i: (0, 0)),
                    pl.BlockSpec((1, d_out), lambda i: (0, 0))],
        out_specs=out_specs,
        out_shape=out_shape,
    )(x, dinv, *p1, *p2, *p3, v, b.reshape(1, -1))
    return res if emit_u else res[0]


def _layer(x, dinv, dinv2, u0, srcq, dstq, w, b, relu, emit_u):
    p1 = _prop_chunks(u0, srcq, dstq)
    u1 = _uscale(_u1_body, dinv2, [p1])
    p2 = _prop_chunks(u1, srcq, dstq)
    u2 = _uscale(_u1_body, dinv2, [p2])
    p3 = _prop_chunks(u2, srcq, dstq)
    return _combine(x, dinv, p1, p2, p3, w, b, relu, emit_u)


def kernel(x, edge_index, W1, b1, W2, b2):
    src = edge_index[0]
    dst = edge_index[1]
    dstq, deg0, deg1 = _degprep_call(src, dst)
    *u0, dinv, dinv2 = _e0(deg0, deg1, x)
    h, *u0p = _layer(x, dinv, dinv2, u0, src, dstq, W1, b1,
                     relu=True, emit_u=True)
    return _layer(h, dinv, dinv2, u0p, src, dstq, W2, b2,
                  relu=False, emit_u=False)


# speedup vs baseline: 1.3227x; 1.0008x over previous
import functools

import jax
import jax.numpy as jnp
from jax import lax
from jax.experimental import pallas as pl
from jax.experimental.pallas import tpu as pltpu
from jax.experimental.pallas import tpu_sc as plsc

_N = 10000
_NP = 10112
_NOUT = 10112
_OPT = 632
_NCORE = 2
_NSUB = 16
_BM = 400
_CB = 96
_CW = 128

_MESH = plsc.VectorSubcoreMesh(core_axis_name="c", subcore_axis_name="s")


def _degprep_body(src_hbm, dst_hbm, dstq_hbm, deg0_hbm, deg1_hbm,
                  src_v, dst_v, dstq_v, isrc, ones_v, zrows, acc):
    c = lax.axis_index("c")
    s = lax.axis_index("s")
    w = s * _NCORE + c
    e = src_hbm.shape[0]
    epw = e // (_NCORE * _NSUB)
    epad = ((epw + 15) // 16) * 16
    ngrp = epad // 16
    iota16 = lax.iota(jnp.int32, 16)
    z16 = jnp.zeros((16,), jnp.float32)
    one16 = jnp.ones((16,), jnp.float32)

    def zrow(r, carry):
        for j in range(_CW // 16):
            zrows[r, pl.ds(j * 16, 16)] = z16
        return carry
    lax.fori_loop(0, 128, zrow, 0)

    def orow(r, carry):
        for j in range(_CW // 16):
            ones_v[r, pl.ds(j * 16, 16)] = one16
        return carry
    lax.fori_loop(0, _CB, orow, 0)

    def zpiece(t, carry):
        pltpu.sync_copy(zrows, acc.at[pl.ds(s * _OPT + t * 128, 128)])
        return carry
    lax.fori_loop(0, 4, zpiece, 0)
    pltpu.sync_copy(zrows.at[pl.ds(0, _OPT - 512)],
                    acc.at[pl.ds(s * _OPT + 512, _OPT - 512)])

    pltpu.sync_copy(src_hbm.at[pl.ds(w * epw, epw)], src_v.at[pl.ds(0, epw)])
    pltpu.sync_copy(dst_hbm.at[pl.ds(w * epw, epw)], dst_v.at[pl.ds(0, epw)])
    plsc.subcore_barrier()

    def grp(i, carry):
        s16 = src_v[pl.ds(i * 16, 16)]
        d16 = dst_v[pl.ds(i * 16, 16)]
        valid = (i * 16 + iota16) < epw
        nosl = (s16 != d16) & valid
        src_v[pl.ds(i * 16, 16)] = jnp.where(nosl, s16, _N)
        dstq_v[pl.ds(i * 16, 16)] = jnp.where(nosl, d16, _N)
        return carry
    lax.fori_loop(0, ngrp, grp, 0)

    pltpu.sync_copy(dstq_v.at[pl.ds(0, epw)], dstq_hbm.at[pl.ds(w * epw, epw)])

    nch = epad // _CB

    def chunk(j, carry):
        for g in range(_CB // 16):
            isrc[pl.ds(g * 16, 16)] = src_v[pl.ds(j * _CB + g * 16, 16)]
        pltpu.sync_copy(ones_v, acc.at[isrc], add=True)
        return carry
    lax.fori_loop(0, nch, chunk, 0)
    rem = epad - nch * _CB
    if rem:
        for g in range(rem // 16):
            isrc[pl.ds(g * 16, 16)] = src_v[pl.ds(nch * _CB + g * 16, 16)]
        for g in range(rem // 16, _CB // 16):
            isrc[pl.ds(g * 16, 16)] = jnp.full((16,), _N, jnp.int32)
        pltpu.sync_copy(ones_v, acc.at[isrc], add=True)

    plsc.subcore_barrier()

    def opiece(t, carry):
        off = s * _OPT + t * 128
        pltpu.sync_copy(acc.at[pl.ds(off, 128)], zrows)

        @pl.when(c == 0)
        def _():
            pltpu.sync_copy(zrows, deg0_hbm.at[pl.ds(off, 128)])

        @pl.when(c == 1)
        def _():
            pltpu.sync_copy(zrows, deg1_hbm.at[pl.ds(off, 128)])
        return carry
    lax.fori_loop(0, 4, opiece, 0)
    lno = _OPT - 512
    offo = s * _OPT + 512
    pltpu.sync_copy(acc.at[pl.ds(offo, lno)], zrows.at[pl.ds(0, lno)])

    @pl.when(c == 0)
    def _():
        pltpu.sync_copy(zrows.at[pl.ds(0, lno)], deg0_hbm.at[pl.ds(offo, lno)])

    @pl.when(c == 1)
    def _():
        pltpu.sync_copy(zrows.at[pl.ds(0, lno)], deg1_hbm.at[pl.ds(offo, lno)])


def _degprep_call(src, dst):
    e = src.shape[0]
    epw = e // (_NCORE * _NSUB)
    epad = ((epw + 15) // 16) * 16
    call = pl.kernel(
        _degprep_body,
        out_type=[jax.ShapeDtypeStruct((e,), jnp.int32),
                  jax.ShapeDtypeStruct((_NOUT, _CW), jnp.float32),
                  jax.ShapeDtypeStruct((_NOUT, _CW), jnp.float32)],
        mesh=_MESH,
        scratch_types=[
            pltpu.VMEM((epad,), jnp.int32),
            pltpu.VMEM((epad,), jnp.int32),
            pltpu.VMEM((epad,), jnp.int32),
            pltpu.VMEM((_CB,), jnp.int32),
            pltpu.VMEM((_CB, _CW), jnp.float32),
            pltpu.VMEM((128, _CW), jnp.float32),
            pltpu.VMEM_SHARED((_NP, _CW), jnp.float32),
        ],
    )
    return call(src, dst)


def _prop_body(g0, g1, srcq, dstq, o0, o1,
               ibs0, ibs1, ibd0, ibd1, isrc_t, idst_t,
               rows0, rows1, rows_t, zrows, acc,
               sem_g0, sem_g1, sem_i0, sem_i1):
    c = lax.axis_index("c")
    s = lax.axis_index("s")
    e = srcq.shape[0]
    ept = e // _NSUB
    base = s * ept
    nfull = ept // _CB
    tail = ept - nfull * _CB

    ibs = (ibs0, ibs1)
    ibd = (ibd0, ibd1)
    rows = (rows0, rows1)
    sem_g = (sem_g0, sem_g1)
    sem_i = (sem_i0, sem_i1)

    z16 = jnp.zeros((16,), jnp.float32)

    def zrow(r, carry):
        for j in range(_CW // 16):
            zrows[r, pl.ds(j * 16, 16)] = z16
        return carry
    lax.fori_loop(0, 128, zrow, 0)

    def zpiece(t, carry):
        pltpu.sync_copy(zrows, acc.at[pl.ds(s * _OPT + t * 128, 128)])
        return carry
    lax.fori_loop(0, 4, zpiece, 0)
    pltpu.sync_copy(zrows.at[pl.ds(0, _OPT - 512)],
                    acc.at[pl.ds(s * _OPT + 512, _OPT - 512)])
    plsc.subcore_barrier()

    def gather_start(p, idxref, rowsref):
        @pl.when(c == 0)
        def _():
            pltpu.async_copy(g0.at[idxref], rowsref, sem_g[p])

        @pl.when(c == 1)
        def _():
            pltpu.async_copy(g1.at[idxref], rowsref, sem_g[p])

    def gather_wait(p, rowsref):
        pltpu.make_async_copy(g0.at[pl.ds(0, rowsref.shape[0])],
                              rowsref, sem_g[p]).wait()

    def scat_start(p):
        pltpu.async_copy(rows[p], acc.at[ibd[p]], sem_i[p], add=True)

    def scat_wait(p):
        pltpu.make_async_copy(rows[p], acc.at[ibd[p]], sem_i[p]).wait()

    pltpu.sync_copy(srcq.at[pl.ds(base, _CB)], ibs0)
    pltpu.sync_copy(dstq.at[pl.ds(base, _CB)], ibd0)

    def half(j, p, first):
        q = 1 - p
        gather_start(p, ibs[p], rows[p])
        if not first:
            scat_wait(q)
        off = base + (j + 1) * _CB
        pltpu.sync_copy(srcq.at[pl.ds(off, _CB)], ibs[q])
        pltpu.sync_copy(dstq.at[pl.ds(off, _CB)], ibd[q])
        gather_wait(p, rows[p])
        scat_start(p)

    half(0, 0, True)

    def duo(j2, carry):
        j = j2 * 2 + 1
        half(j, 1, False)
        half(j + 1, 0, False)
        return carry
    lax.fori_loop(0, (nfull - 2) // 2, duo, 0)
    gather_start(1, ibs1, rows1)
    scat_wait(0)
    gather_wait(1, rows1)
    scat_start(1)
    scat_wait(1)

    if tail:
        off = base + nfull * _CB
        pltpu.sync_copy(srcq.at[pl.ds(off, tail)], isrc_t)
        pltpu.sync_copy(dstq.at[pl.ds(off, tail)], idst_t)
        gather_start(0, isrc_t, rows_t)
        pltpu.make_async_copy(g0.at[pl.ds(0, tail)], rows_t, sem_g0).wait()
        pltpu.sync_copy(rows_t, acc.at[idst_t], add=True)

    plsc.subcore_barrier()

    def opiece(t, carry):
        off = s * _OPT + t * 128
        pltpu.sync_copy(acc.at[pl.ds(off, 128)], zrows)

        @pl.when(c == 0)
        def _():
            pltpu.sync_copy(zrows, o0.at[pl.ds(off, 128)])

        @pl.when(c == 1)
        def _():
            pltpu.sync_copy(zrows, o1.at[pl.ds(off, 128)])
        return carry
    lax.fori_loop(0, 4, opiece, 0)
    lno = _OPT - 512
    offo = s * _OPT + 512
    pltpu.sync_copy(acc.at[pl.ds(offo, lno)], zrows.at[pl.ds(0, lno)])

    @pl.when(c == 0)
    def _():
        pltpu.sync_copy(zrows.at[pl.ds(0, lno)], o0.at[pl.ds(offo, lno)])

    @pl.when(c == 1)
    def _():
        pltpu.sync_copy(zrows.at[pl.ds(0, lno)], o1.at[pl.ds(offo, lno)])


def _prop_pair(g0, g1, srcq, dstq):
    e = srcq.shape[0]
    ept = e // _NSUB
    tail = ept - (ept // _CB) * _CB
    call = pl.kernel(
        _prop_body,
        out_type=[jax.ShapeDtypeStruct((_NOUT, _CW), jnp.float32),
                  jax.ShapeDtypeStruct((_NOUT, _CW), jnp.float32)],
        mesh=_MESH,
        scratch_types=[
            pltpu.VMEM((_CB,), jnp.int32),
            pltpu.VMEM((_CB,), jnp.int32),
            pltpu.VMEM((_CB,), jnp.int32),
            pltpu.VMEM((_CB,), jnp.int32),
            pltpu.VMEM((max(tail, 8),), jnp.int32),
            pltpu.VMEM((max(tail, 8),), jnp.int32),
            pltpu.VMEM((_CB, _CW), jnp.float32),
            pltpu.VMEM((_CB, _CW), jnp.float32),
            pltpu.VMEM((max(tail, 8), _CW), jnp.float32),
            pltpu.VMEM((128, _CW), jnp.float32),
            pltpu.VMEM_SHARED((_NP, _CW), jnp.float32),
            pltpu.SemaphoreType.DMA,
            pltpu.SemaphoreType.DMA,
            pltpu.SemaphoreType.DMA,
            pltpu.SemaphoreType.DMA,
        ],
    )
    return call(g0, g1, srcq, dstq)


def _prop_chunks(chunks, srcq, dstq):
    out = []
    for i in range(0, len(chunks), 2):
        o0, o1 = _prop_pair(chunks[i], chunks[i + 1], srcq, dstq)
        out.extend([o0, o1])
    return out


def _e0_body(deg0_ref, deg1_ref, x_ref, *out_refs):
    d = deg0_ref[:, 0:1] + deg1_ref[:, 0:1]
    dinv = jnp.where(d > 0, lax.rsqrt(d), 0.0)
    nc = x_ref.shape[1] // _CW
    for cch in range(nc):
        out_refs[cch][...] = dinv * x_ref[:, cch * _CW:(cch + 1) * _CW]
    out_refs[nc][...] = dinv
    out_refs[nc + 1][...] = dinv * dinv


def _e0(deg0, deg1, x):
    n = x.shape[0]
    nc = x.shape[1] // _CW
    grid = (n // _BM,)
    cspec = pl.BlockSpec((_BM, _CW), lambda i: (i, 0))
    return pl.pallas_call(
        _e0_body,
        grid=grid,
        in_specs=[pl.BlockSpec((_BM, _CW), lambda i: (i, 0)),
                  pl.BlockSpec((_BM, _CW), lambda i: (i, 0)),
                  pl.BlockSpec((_BM, x.shape[1]), lambda i: (i, 0))],
        out_specs=[cspec] * nc + [pl.BlockSpec((_BM, 1), lambda i: (i, 0))] * 2,
        out_shape=[jax.ShapeDtypeStruct((n, _CW), jnp.float32)] * nc
                  + [jax.ShapeDtypeStruct((n, 1), jnp.float32)] * 2,
    )(deg0, deg1, x)


def _u1_body(nc, d2_ref, *refs):
    d2 = d2_ref[...]
    for c in range(nc):
        refs[nc + c][...] = -d2 * refs[c][...]


def _uscale(body, d2, chunk_lists):
    nc = len(chunk_lists[0])
    n = d2.shape[0]
    grid = (n // _BM,)
    cspec = pl.BlockSpec((_BM, _CW), lambda i: (i, 0))
    flat = [a for lst in chunk_lists for a in lst]
    return pl.pallas_call(
        functools.partial(body, nc),
        grid=grid,
        in_specs=[pl.BlockSpec((_BM, 1), lambda i: (i, 0))] + [cspec] * len(flat),
        out_specs=[cspec] * nc,
        out_shape=[jax.ShapeDtypeStruct((n, _CW), jnp.float32)] * nc,
    )(d2, *flat)


def _combine_body(nc, d_in, relu, emit_u, x_ref, dinv_ref, *refs):
    dinv = dinv_ref[...]
    parts = [x_ref[...]]
    for i in range(3 * nc):
        parts.append(dinv * refs[i][...])
    a = jnp.concatenate(parts, axis=1)
    v_ref = refs[3 * nc]
    b_ref = refs[3 * nc + 1]
    y = jnp.dot(a, v_ref[...], preferred_element_type=jnp.float32) + b_ref[...]
    if relu:
        y = jnp.maximum(y, 0.0)
    refs[3 * nc + 2][...] = y
    if emit_u:
        for cch in range(y.shape[1] // _CW):
            refs[3 * nc + 3 + cch][...] = dinv * y[:, cch * _CW:(cch + 1) * _CW]


def _combine(x, dinv, p1, p2, p3, w, b, relu, emit_u):
    n, d_in = x.shape
    d_out = w.shape[2]
    nc = d_in // _CW
    v = jnp.concatenate([w[0] - w[2], 3.0 * w[3] - w[1],
                         -2.0 * w[2], -4.0 * w[3]], axis=0)
    grid = (n // _BM,)
    cspec = pl.BlockSpec((_BM, _CW), lambda i: (i, 0))
    out_shape = [jax.ShapeDtypeStruct((n, d_out), jnp.float32)]
    out_specs = [pl.BlockSpec((_BM, d_out), lambda i: (i, 0))]
    if emit_u:
        out_shape += [jax.ShapeDtypeStruct((n, _CW), jnp.float32)] * (d_out // _CW)
        out_specs += [cspec] * (d_out // _CW)
    res = pl.pallas_call(
        functools.partial(_combine_body, nc, d_in, relu, emit_u),
        grid=grid,
        in_specs=[pl.BlockSpec((_BM, d_in), lambda i: (i, 0)),
                  pl.BlockSpec((_BM, 1), lambda i: (i, 0))]
                 + [cspec] * (3 * nc)
                 + [pl.BlockSpec((4 * d_in, d_out), lambda i: (0, 0)),
                    pl.BlockSpec((1, d_out), lambda i: (0, 0))],
        out_specs=out_specs,
        out_shape=out_shape,
    )(x, dinv, *p1, *p2, *p3, v, b.reshape(1, -1))
    return res if emit_u else res[0]


def _layer(x, dinv, dinv2, u0, srcq, dstq, w, b, relu, emit_u):
    p1 = _prop_chunks(u0, srcq, dstq)
    u1 = _uscale(_u1_body, dinv2, [p1])
    p2 = _prop_chunks(u1, srcq, dstq)
    u2 = _uscale(_u1_body, dinv2, [p2])
    p3 = _prop_chunks(u2, srcq, dstq)
    return _combine(x, dinv, p1, p2, p3, w, b, relu, emit_u)


def kernel(x, edge_index, W1, b1, W2, b2):
    src = edge_index[0]
    dst = edge_index[1]
    dstq, deg0, deg1 = _degprep_call(src, dst)
    *u0, dinv, dinv2 = _e0(deg0, deg1, x)
    h, *u0p = _layer(x, dinv, dinv2, u0, src, dstq, W1, b1,
                     relu=True, emit_u=True)
    return _layer(h, dinv, dinv2, u0p, src, dstq, W2, b2,
                  relu=False, emit_u=False)
